# Initial kernel scaffold; baseline (speedup 1.0000x reference)
import jax
import jax.numpy as jnp
from jax.experimental import pallas as pl


def kernel(x, edge_index, eps):
    raise NotImplementedError("write your pallas kernel here")


# speedup vs baseline: 27.6044x; 27.6044x over previous
import functools

import jax
import jax.numpy as jnp
from jax import lax
from jax.experimental import pallas as pl
from jax.experimental.pallas import tpu as pltpu
from jax.experimental.pallas import tpu_sc as plsc

N = 10000
E = 320000
D = 128
NT = 32
EPT = E // NT
C = 125
NCH = EPT // C
NBUF = 4
N_PAD = 10240
RPS = N_PAD // 16

_mesh = plsc.VectorSubcoreMesh(core_axis_name="c", subcore_axis_name="s")


@functools.partial(
    pl.kernel,
    out_type=jax.ShapeDtypeStruct((NT, N_PAD // 128, 128), jnp.float32),
    mesh=_mesh,
    compiler_params=pltpu.CompilerParams(needs_layout_passes=False),
    scratch_types=[
        pltpu.VMEM((EPT,), jnp.int32),
        pltpu.VMEM((N_PAD // 128, 128), jnp.float32),
    ],
)
def _hist_sc(col_hbm, out_hbm, colbuf, hist):
    c = lax.axis_index("c")
    s = lax.axis_index("s")
    w = s * 2 + c
    pltpu.sync_copy(col_hbm.at[pl.ds(w * EPT, EPT)], colbuf)

    zeros16 = jnp.zeros((16,), jnp.float32)
    ones16 = jnp.ones((16,), jnp.float32)

    def zbody(i, carry):
        r = i // 8
        k = i % 8
        hist[r, pl.ds(k * 16, 16)] = zeros16
        return carry

    lax.fori_loop(0, N_PAD // 16, zbody, 0)

    def hbody(i, carry):
        idx = colbuf[pl.ds(i * 16, 16)]
        plsc.addupdate_scatter(
            hist, [lax.shift_right_logical(idx, 7), lax.bitwise_and(idx, 127)],
            ones16)
        return carry

    lax.fori_loop(0, EPT // 16, hbody, 0)
    pltpu.sync_copy(hist, out_hbm.at[w])


def _dinv_body(h_ref, o_ref):
    deg = jnp.sum(h_ref[...], axis=0, keepdims=True)
    o_ref[...] = jnp.where(deg > 0.0, lax.rsqrt(deg), 0.0)


def _dinv_tc(hist_parts):
    return pl.pallas_call(
        _dinv_body,
        out_shape=jax.ShapeDtypeStruct((1, N_PAD), jnp.float32),
    )(hist_parts)


def _scale_body(x_ref, d_ref, o_ref):
    o_ref[...] = x_ref[...] * d_ref[...]


def _scale_tc(x, dinv_col):
    bs = 1000
    return pl.pallas_call(
        _scale_body,
        grid=(N // bs,),
        in_specs=[
            pl.BlockSpec((bs, D), lambda i: (i, 0)),
            pl.BlockSpec((bs, 1), lambda i: (i, 0)),
        ],
        out_specs=pl.BlockSpec((bs, D), lambda i: (i, 0)),
        out_shape=jax.ShapeDtypeStruct((N, D), jnp.float32),
    )(x, dinv_col)


DH = 64


def _spmm_body(xp_hbm, col_hbm, row_hbm, out_hbm,
               col_v, row_v, bufs, zblk, agg_sh, gsem, ssem):
    c = lax.axis_index("c")
    s = lax.axis_index("s")
    w = s * 2 + c

    pltpu.sync_copy(col_hbm.at[w], col_v)
    pltpu.sync_copy(row_hbm.at[w], row_v)

    zeros16 = jnp.zeros((16,), jnp.float32)

    def zbody(i, carry):
        r = i // (DH // 16)
        k = i % (DH // 16)
        zblk[r, pl.ds(k * 16, 16)] = zeros16
        return carry

    lax.fori_loop(0, 128 * (DH // 16), zbody, 0)

    def zcpy(k, carry):
        pltpu.sync_copy(zblk, agg_sh.at[pl.ds(s * RPS + k * 128, 128)])
        return carry

    lax.fori_loop(0, RPS // 128, zcpy, 0)
    plsc.subcore_barrier()

    def _gather(jj, b):
        return pltpu.async_copy(xp_hbm.at[col_v.at[jj]], bufs.at[b],
                                gsem.at[b])

    def _gather_wait(jj, b):
        pltpu.make_async_copy(xp_hbm.at[col_v.at[jj]], bufs.at[b],
                              gsem.at[b]).wait()

    def _scatter(jj, b):
        return pltpu.async_copy(bufs.at[b], agg_sh.at[row_v.at[jj]],
                                ssem.at[b], add=True)

    def _scatter_wait(jj, b):
        pltpu.make_async_copy(bufs.at[b], agg_sh.at[row_v.at[jj]],
                              ssem.at[b]).wait()

    for b in range(NBUF):
        _gather(b, b)

    def mbody(jo, carry):
        for b in range(NBUF):
            jj = jo * NBUF + b
            _gather_wait(jj, b)
            _scatter(jj, b)
            _scatter_wait(jj, b)
            _gather(jj + NBUF, b)
        return carry

    lax.fori_loop(0, NCH // NBUF - 1, mbody, 0)
    for b in range(NBUF):
        jj = NCH - NBUF + b
        _gather_wait(jj, b)
        _scatter(jj, b)
        _scatter_wait(jj, b)

    plsc.subcore_barrier()

    def wbody(k, carry):
        base = s * RPS + k * 128
        pltpu.sync_copy(agg_sh.at[pl.ds(base, 128)], zblk)
        pltpu.sync_copy(zblk, out_hbm.at[c, pl.ds(base, 128)])
        return carry

    lax.fori_loop(0, RPS // 128, wbody, 0)


_spmm_sc = pl.kernel(
    _spmm_body,
    out_type=jax.ShapeDtypeStruct((2, N_PAD, DH), jnp.float32),
    mesh=_mesh,
    compiler_params=pltpu.CompilerParams(use_tc_tiling_on_sc=False),
    scratch_types=[
        pltpu.VMEM((NCH, C), jnp.int32),
        pltpu.VMEM((NCH, C), jnp.int32),
        pltpu.VMEM((NBUF, C, DH), jnp.float32),
        pltpu.VMEM((128, DH), jnp.float32),
        pltpu.VMEM_SHARED((N_PAD, DH), jnp.float32),
        pltpu.SemaphoreType.DMA((NBUF,)),
        pltpu.SemaphoreType.DMA((NBUF,)),
    ],
)


def _comb_body(pa0_ref, pa1_ref, pb0_ref, pb1_ref, x_ref, d_ref, e_ref,
               o_ref):
    d = d_ref[...]
    scale = 1.0 + e_ref[0, 0]
    lo = d * (pa0_ref[...] + pa1_ref[...])
    hi = d * (pb0_ref[...] + pb1_ref[...])
    o_ref[...] = jnp.concatenate([lo, hi], axis=1) + scale * x_ref[...]


def _comb_tc(pa0, pa1, pb0, pb1, x, dinv_col, eps2d):
    bs = 1000
    half = pl.BlockSpec((bs, DH), lambda i: (i, 0))
    return pl.pallas_call(
        _comb_body,
        grid=(N // bs,),
        in_specs=[
            half, half, half, half,
            pl.BlockSpec((bs, D), lambda i: (i, 0)),
            pl.BlockSpec((bs, 1), lambda i: (i, 0)),
            pl.BlockSpec((1, 1), lambda i: (0, 0)),
        ],
        out_specs=pl.BlockSpec((bs, D), lambda i: (i, 0)),
        out_shape=jax.ShapeDtypeStruct((N, D), jnp.float32),
    )(pa0, pa1, pb0, pb1, x, dinv_col, eps2d)


def kernel(x, edge_index, eps):
    ei = edge_index.astype(jnp.int32)
    row = ei[0]
    col = ei[1]

    hist_parts = _hist_sc(col).reshape(NT, N_PAD)
    dinv2d = _dinv_tc(hist_parts)
    dinv_col = dinv2d.reshape(N_PAD)[:N, None]
    xp = _scale_tc(x, dinv_col)

    col_t = col.reshape(NT, NCH, C)
    row_t = row.reshape(NT, NCH, C)
    parts_a = _spmm_sc(xp[:, :DH], col_t, row_t)
    parts_b = _spmm_sc(xp[:, DH:], col_t, row_t)

    return _comb_tc(parts_a[0, :N], parts_a[1, :N],
                    parts_b[0, :N], parts_b[1, :N],
                    x, dinv_col, eps.reshape(1, 1))


# speedup vs baseline: 30.0492x; 1.0886x over previous
import functools

import jax
import jax.numpy as jnp
from jax import lax
from jax.experimental import pallas as pl
from jax.experimental.pallas import tpu as pltpu
from jax.experimental.pallas import tpu_sc as plsc

N = 10000
E = 320000
D = 128
NT = 32
EPT = E // NT
C = 125
NCH = EPT // C
NBUF = 4
N_PAD = 10240
RPS = N_PAD // 16

_mesh = plsc.VectorSubcoreMesh(core_axis_name="c", subcore_axis_name="s")


@functools.partial(
    pl.kernel,
    out_type=jax.ShapeDtypeStruct((NT, N_PAD // 128, 128), jnp.float32),
    mesh=_mesh,
    compiler_params=pltpu.CompilerParams(needs_layout_passes=False),
    scratch_types=[
        pltpu.VMEM((EPT,), jnp.int32),
        pltpu.VMEM((N_PAD // 128, 128), jnp.float32),
    ],
)
def _hist_sc(col_hbm, out_hbm, colbuf, hist):
    c = lax.axis_index("c")
    s = lax.axis_index("s")
    w = s * 2 + c
    pltpu.sync_copy(col_hbm.at[pl.ds(w * EPT, EPT)], colbuf)

    zeros16 = jnp.zeros((16,), jnp.float32)
    ones16 = jnp.ones((16,), jnp.float32)

    def zbody(i, carry):
        r = i // 8
        k = i % 8
        hist[r, pl.ds(k * 16, 16)] = zeros16
        return carry

    lax.fori_loop(0, N_PAD // 16, zbody, 0)

    def hbody(i, carry):
        idx = colbuf[pl.ds(i * 16, 16)]
        plsc.addupdate_scatter(
            hist, [lax.shift_right_logical(idx, 7), lax.bitwise_and(idx, 127)],
            ones16)
        return carry

    lax.fori_loop(0, EPT // 16, hbody, 0)
    pltpu.sync_copy(hist, out_hbm.at[w])


def _dinv_body(h_ref, o_ref):
    deg = jnp.sum(h_ref[...], axis=0, keepdims=True)
    o_ref[...] = jnp.where(deg > 0.0, lax.rsqrt(deg), 0.0)


def _dinv_tc(hist_parts):
    return pl.pallas_call(
        _dinv_body,
        out_shape=jax.ShapeDtypeStruct((1, N_PAD), jnp.float32),
    )(hist_parts)


def _scale_body(x_ref, d_ref, lo_ref, hi_ref):
    xs = x_ref[...] * d_ref[...]
    lo_ref[...] = xs[:, :DH]
    hi_ref[...] = xs[:, DH:]


def _scale_tc(x, dinv_col):
    bs = 1000
    return pl.pallas_call(
        _scale_body,
        grid=(N // bs,),
        in_specs=[
            pl.BlockSpec((bs, D), lambda i: (i, 0)),
            pl.BlockSpec((bs, 1), lambda i: (i, 0)),
        ],
        out_specs=[
            pl.BlockSpec((bs, DH), lambda i: (i, 0)),
            pl.BlockSpec((bs, DH), lambda i: (i, 0)),
        ],
        out_shape=[
            jax.ShapeDtypeStruct((N, DH), jnp.float32),
            jax.ShapeDtypeStruct((N, DH), jnp.float32),
        ],
    )(x, dinv_col)


DH = 64


def _spmm_body(xp_hbm, col_hbm, row_hbm, out_hbm,
               col_v, row_v, bufs, zblk, agg_sh, gsem, ssem):
    c = lax.axis_index("c")
    s = lax.axis_index("s")
    w = s * 2 + c

    pltpu.sync_copy(col_hbm.at[w], col_v)
    pltpu.sync_copy(row_hbm.at[w], row_v)

    zeros16 = jnp.zeros((16,), jnp.float32)

    def zbody(i, carry):
        r = i // (DH // 16)
        k = i % (DH // 16)
        zblk[r, pl.ds(k * 16, 16)] = zeros16
        return carry

    lax.fori_loop(0, 128 * (DH // 16), zbody, 0)

    def zcpy(k, carry):
        pltpu.sync_copy(zblk, agg_sh.at[pl.ds(s * RPS + k * 128, 128)])
        return carry

    lax.fori_loop(0, RPS // 128, zcpy, 0)
    plsc.subcore_barrier()

    def _gather(jj, b):
        return pltpu.async_copy(xp_hbm.at[col_v.at[jj]], bufs.at[b],
                                gsem.at[b])

    def _gather_wait(jj, b):
        pltpu.make_async_copy(xp_hbm.at[col_v.at[jj]], bufs.at[b],
                              gsem.at[b]).wait()

    def _scatter(jj, b):
        return pltpu.async_copy(bufs.at[b], agg_sh.at[row_v.at[jj]],
                                ssem.at[b], add=True)

    def _scatter_wait(jj, b):
        pltpu.make_async_copy(bufs.at[b], agg_sh.at[row_v.at[jj]],
                              ssem.at[b]).wait()

    for b in range(NBUF):
        _gather(b, b)

    def mbody(jo, carry):
        for b in range(NBUF):
            jj = jo * NBUF + b
            _gather_wait(jj, b)
            _scatter(jj, b)
            _scatter_wait(jj, b)
            _gather(jj + NBUF, b)
        return carry

    lax.fori_loop(0, NCH // NBUF - 1, mbody, 0)
    for b in range(NBUF):
        jj = NCH - NBUF + b
        _gather_wait(jj, b)
        _scatter(jj, b)
        _scatter_wait(jj, b)

    plsc.subcore_barrier()

    def wbody(k, carry):
        base = s * RPS + k * 128
        pltpu.sync_copy(agg_sh.at[pl.ds(base, 128)], zblk)
        pltpu.sync_copy(zblk, out_hbm.at[c, pl.ds(base, 128)])
        return carry

    lax.fori_loop(0, RPS // 128, wbody, 0)


_spmm_sc = pl.kernel(
    _spmm_body,
    out_type=jax.ShapeDtypeStruct((2, N_PAD, DH), jnp.float32),
    mesh=_mesh,
    compiler_params=pltpu.CompilerParams(use_tc_tiling_on_sc=False),
    scratch_types=[
        pltpu.VMEM((NCH, C), jnp.int32),
        pltpu.VMEM((NCH, C), jnp.int32),
        pltpu.VMEM((NBUF, C, DH), jnp.float32),
        pltpu.VMEM((128, DH), jnp.float32),
        pltpu.VMEM_SHARED((N_PAD, DH), jnp.float32),
        pltpu.SemaphoreType.DMA((NBUF,)),
        pltpu.SemaphoreType.DMA((NBUF,)),
    ],
)


def _comb_body(pa0_ref, pa1_ref, pb0_ref, pb1_ref, x_ref, d_ref, e_ref,
               o_ref):
    d = d_ref[...]
    scale = 1.0 + e_ref[0, 0]
    lo = d * (pa0_ref[0] + pa1_ref[0])
    hi = d * (pb0_ref[0] + pb1_ref[0])
    o_ref[...] = jnp.concatenate([lo, hi], axis=1) + scale * x_ref[...]


def _comb_tc(parts_a, parts_b, x, dinv_col, eps2d):
    bs = 1000
    c0 = pl.BlockSpec((1, bs, DH), lambda i: (0, i, 0))
    c1 = pl.BlockSpec((1, bs, DH), lambda i: (1, i, 0))
    return pl.pallas_call(
        _comb_body,
        grid=(N // bs,),
        in_specs=[
            c0, c1, c0, c1,
            pl.BlockSpec((bs, D), lambda i: (i, 0)),
            pl.BlockSpec((bs, 1), lambda i: (i, 0)),
            pl.BlockSpec((1, 1), lambda i: (0, 0)),
        ],
        out_specs=pl.BlockSpec((bs, D), lambda i: (i, 0)),
        out_shape=jax.ShapeDtypeStruct((N, D), jnp.float32),
    )(parts_a, parts_a, parts_b, parts_b, x, dinv_col, eps2d)


def kernel(x, edge_index, eps):
    ei = edge_index.astype(jnp.int32)
    row = ei[0]
    col = ei[1]

    hist_parts = _hist_sc(col).reshape(NT, N_PAD)
    dinv2d = _dinv_tc(hist_parts)
    dinv_col = dinv2d.reshape(N_PAD)[:N, None]
    xp_lo, xp_hi = _scale_tc(x, dinv_col)

    col_t = col.reshape(NT, NCH, C)
    row_t = row.reshape(NT, NCH, C)
    parts_a = _spmm_sc(xp_lo, col_t, row_t)
    parts_b = _spmm_sc(xp_hi, col_t, row_t)

    return _comb_tc(parts_a, parts_b, x, dinv_col, eps.reshape(1, 1))


# speedup vs baseline: 31.3034x; 1.0417x over previous
import functools

import jax
import jax.numpy as jnp
from jax import lax
from jax.experimental import pallas as pl
from jax.experimental.pallas import tpu as pltpu
from jax.experimental.pallas import tpu_sc as plsc

N = 10000
E = 320000
D = 128
NT = 32
EPT = E // NT
C = 125
NCH = EPT // C
NBUF = 4
N_PAD = 10240
RPS = N_PAD // 16

_mesh = plsc.VectorSubcoreMesh(core_axis_name="c", subcore_axis_name="s")


EPH = E // 16


@functools.partial(
    pl.kernel,
    out_type=jax.ShapeDtypeStruct((16, 5, 128), jnp.float32),
    mesh=_mesh,
    compiler_params=pltpu.CompilerParams(needs_layout_passes=False),
    scratch_types=[
        pltpu.VMEM((EPH,), jnp.int32),
        pltpu.VMEM((N_PAD // 128, 128), jnp.float32),
        pltpu.VMEM((5, 128), jnp.float32),
        pltpu.VMEM((N_PAD // 128,), jnp.int32),
        pltpu.VMEM_SHARED((N_PAD // 128, 128), jnp.float32),
    ],
)
def _hist_sc(ei_hbm, out_hbm, colbuf, hist, zblk5, idrow, hist_sh):
    c = lax.axis_index("c")
    s = lax.axis_index("s")

    @pl.when(c == 0)
    def _():
        pltpu.sync_copy(ei_hbm.at[pl.ds(E + s * EPH, EPH)], colbuf)

        zeros16 = jnp.zeros((16,), jnp.float32)
        ones16 = jnp.ones((16,), jnp.float32)
        iota16 = lax.iota(jnp.int32, 16)

        def zbody(i, carry):
            r = i // 8
            k = i % 8
            hist[r, pl.ds(k * 16, 16)] = zeros16
            return carry

        lax.fori_loop(0, N_PAD // 16, zbody, 0)
        for r in range(5):
            for k in range(8):
                zblk5[r, pl.ds(k * 16, 16)] = zeros16
        for k in range(5):
            idrow[pl.ds(k * 16, 16)] = iota16 + k * 16

        def hbody(i, carry):
            idx = colbuf[pl.ds(i * 16, 16)]
            plsc.addupdate_scatter(
                hist,
                [lax.shift_right_logical(idx, 7), lax.bitwise_and(idx, 127)],
                ones16)
            return carry

        lax.fori_loop(0, EPH // 16, hbody, 0)

        pltpu.sync_copy(zblk5, hist_sh.at[pl.ds(s * 5, 5)])
        plsc.subcore_barrier()
        pltpu.sync_copy(hist, hist_sh.at[idrow], add=True)
        plsc.subcore_barrier()

        pltpu.sync_copy(hist_sh.at[pl.ds(s * 5, 5)], zblk5)
        pltpu.sync_copy(zblk5, out_hbm.at[s])


def _dinv_of(deg):
    return jnp.where(deg > 0.0, lax.rsqrt(deg), 0.0)


def _scale_body(x_ref, d_ref, lo_ref, hi_ref):
    xs = x_ref[...] * _dinv_of(d_ref[...])
    lo_ref[...] = xs[:, :DH]
    hi_ref[...] = xs[:, DH:]


def _scale_tc(x, deg_col):
    bs = 1000
    return pl.pallas_call(
        _scale_body,
        grid=(N // bs,),
        in_specs=[
            pl.BlockSpec((bs, D), lambda i: (i, 0)),
            pl.BlockSpec((bs, 1), lambda i: (i, 0)),
        ],
        out_specs=[
            pl.BlockSpec((bs, DH), lambda i: (i, 0)),
            pl.BlockSpec((bs, DH), lambda i: (i, 0)),
        ],
        out_shape=[
            jax.ShapeDtypeStruct((N, DH), jnp.float32),
            jax.ShapeDtypeStruct((N, DH), jnp.float32),
        ],
    )(x, deg_col)


DH = 64


def _spmm_body(xp_hbm, ei_hbm, out_hbm,
               col_v, row_v, bufs, zblk, agg_sh, gsem, ssem):
    c = lax.axis_index("c")
    s = lax.axis_index("s")
    w = s * 2 + c

    pltpu.sync_copy(ei_hbm.at[NT + w], col_v)
    pltpu.sync_copy(ei_hbm.at[w], row_v)

    zeros16 = jnp.zeros((16,), jnp.float32)

    def zbody(i, carry):
        r = i // (DH // 16)
        k = i % (DH // 16)
        zblk[r, pl.ds(k * 16, 16)] = zeros16
        return carry

    lax.fori_loop(0, 128 * (DH // 16), zbody, 0)

    def zcpy(k, carry):
        pltpu.sync_copy(zblk, agg_sh.at[pl.ds(s * RPS + k * 128, 128)])
        return carry

    lax.fori_loop(0, RPS // 128, zcpy, 0)
    plsc.subcore_barrier()

    def _gather(jj, b):
        return pltpu.async_copy(xp_hbm.at[col_v.at[jj]], bufs.at[b],
                                gsem.at[b])

    def _gather_wait(jj, b):
        pltpu.make_async_copy(xp_hbm.at[col_v.at[jj]], bufs.at[b],
                              gsem.at[b]).wait()

    def _scatter(jj, b):
        return pltpu.async_copy(bufs.at[b], agg_sh.at[row_v.at[jj]],
                                ssem.at[b], add=True)

    def _scatter_wait(jj, b):
        pltpu.make_async_copy(bufs.at[b], agg_sh.at[row_v.at[jj]],
                              ssem.at[b]).wait()

    for b in range(NBUF):
        _gather(b, b)

    def mbody(jo, carry):
        for b in range(NBUF):
            jj = jo * NBUF + b
            _gather_wait(jj, b)
            _scatter(jj, b)
            _scatter_wait(jj, b)
            _gather(jj + NBUF, b)
        return carry

    lax.fori_loop(0, NCH // NBUF - 1, mbody, 0)
    for b in range(NBUF):
        jj = NCH - NBUF + b
        _gather_wait(jj, b)
        _scatter(jj, b)
        _scatter_wait(jj, b)

    plsc.subcore_barrier()

    def wbody(k, carry):
        base = s * RPS + k * 128
        pltpu.sync_copy(agg_sh.at[pl.ds(base, 128)], zblk)
        pltpu.sync_copy(zblk, out_hbm.at[c, pl.ds(base, 128)])
        return carry

    lax.fori_loop(0, RPS // 128, wbody, 0)


_spmm_sc = pl.kernel(
    _spmm_body,
    out_type=jax.ShapeDtypeStruct((2, N_PAD, DH), jnp.float32),
    mesh=_mesh,
    compiler_params=pltpu.CompilerParams(use_tc_tiling_on_sc=False),
    scratch_types=[
        pltpu.VMEM((NCH, C), jnp.int32),
        pltpu.VMEM((NCH, C), jnp.int32),
        pltpu.VMEM((NBUF, C, DH), jnp.float32),
        pltpu.VMEM((128, DH), jnp.float32),
        pltpu.VMEM_SHARED((N_PAD, DH), jnp.float32),
        pltpu.SemaphoreType.DMA((NBUF,)),
        pltpu.SemaphoreType.DMA((NBUF,)),
    ],
)


def _comb_body(pa0_ref, pa1_ref, pb0_ref, pb1_ref, x_ref, d_ref, e_ref,
               o_ref):
    d = _dinv_of(d_ref[...])
    scale = 1.0 + e_ref[0, 0]
    lo = d * (pa0_ref[0] + pa1_ref[0])
    hi = d * (pb0_ref[0] + pb1_ref[0])
    o_ref[...] = jnp.concatenate([lo, hi], axis=1) + scale * x_ref[...]


def _comb_tc(parts_a, parts_b, x, dinv_col, eps2d):
    bs = 1000
    c0 = pl.BlockSpec((1, bs, DH), lambda i: (0, i, 0))
    c1 = pl.BlockSpec((1, bs, DH), lambda i: (1, i, 0))
    return pl.pallas_call(
        _comb_body,
        grid=(N // bs,),
        in_specs=[
            c0, c1, c0, c1,
            pl.BlockSpec((bs, D), lambda i: (i, 0)),
            pl.BlockSpec((bs, 1), lambda i: (i, 0)),
            pl.BlockSpec((1, 1), lambda i: (0, 0)),
        ],
        out_specs=pl.BlockSpec((bs, D), lambda i: (i, 0)),
        out_shape=jax.ShapeDtypeStruct((N, D), jnp.float32),
    )(parts_a, parts_a, parts_b, parts_b, x, dinv_col, eps2d)


def kernel(x, edge_index, eps):
    ei = edge_index.astype(jnp.int32)

    deg_col = _hist_sc(ei.reshape(2 * E)).reshape(N_PAD, 1)
    xp_lo, xp_hi = _scale_tc(x, deg_col)

    ei3 = ei.reshape(2 * NT, NCH, C)
    parts_a = _spmm_sc(xp_lo, ei3)
    parts_b = _spmm_sc(xp_hi, ei3)

    return _comb_tc(parts_a, parts_b, x, deg_col, eps.reshape(1, 1))
